# baseline (speedup 1.0000x reference)
import functools
import jax
import jax.numpy as jnp
from jax import lax
from jax.experimental import pallas as pl
from jax.experimental.pallas import tpu as pltpu
from jax.experimental.pallas import tpu_sc as plsc

B, L, D = 1024, 512, 128
B_SC = 384
B_TC = B - B_SC
RG = 4
CHUNK_ROWS = L // RG
NC, NS = 2, 16
NW = NC * NS
BGROUPS = NW // RG
BPW = B_SC // BGROUPS
NBUF = 4

_mesh = plsc.VectorSubcoreMesh(core_axis_name="c", subcore_axis_name="s")


@functools.partial(
    pl.kernel,
    mesh=_mesh,
    out_type=jax.ShapeDtypeStruct((B_SC * RG, CHUNK_ROWS, D), jnp.float32),
    scratch_types=(
        [pltpu.VMEM((CHUNK_ROWS, D), jnp.float32)]
        + [pltpu.VMEM((CHUNK_ROWS, D), jnp.float32) for _ in range(NBUF)]
        + [pltpu.SemaphoreType.DMA for _ in range(2 * NBUF)]
    ),
)
def _sc_add(embed_hbm, pos_hbm, out_hbm, pos_v, b0, b1, b2, b3,
            si0, si1, si2, si3, so0, so1, so2, so3):
    bufs = (b0, b1, b2, b3)
    in_sems = (si0, si1, si2, si3)
    out_sems = (so0, so1, so2, so3)

    wid = lax.axis_index("s") * NC + lax.axis_index("c")
    bg = wid // RG
    rg = wid % RG
    base = bg * BPW

    def chunk_idx(k):
        return (base + k) * RG + rg

    pltpu.sync_copy(pos_hbm.at[rg], pos_v)

    pltpu.async_copy(embed_hbm.at[chunk_idx(0)], bufs[0], in_sems[0])
    pltpu.async_copy(embed_hbm.at[chunk_idx(1)], bufs[1], in_sems[1])

    def group(g, carry):
        for s in range(NBUF):
            k = g * NBUF + s
            buf = bufs[s]
            c = chunk_idx(k)
            pltpu.make_async_copy(embed_hbm.at[c], buf, in_sems[s]).wait()

            def add_body(r, carry2):
                for j in range(D // 16):
                    sl = pl.ds(j * 16, 16)
                    plsc.addupdate(buf.at[r, sl], pos_v[r, sl])
                return carry2

            lax.fori_loop(0, CHUNK_ROWS, add_body, 0)

            pltpu.async_copy(buf, out_hbm.at[c], out_sems[s])

            s2 = (s + 2) % NBUF
            if s < 2:
                @pl.when(g >= 1)
                def _():
                    pltpu.make_async_copy(
                        bufs[s2], out_hbm.at[chunk_idx(k - 2)], out_sems[s2]
                    ).wait()

                pltpu.async_copy(
                    embed_hbm.at[chunk_idx(k + 2)], bufs[s2], in_sems[s2]
                )
            else:
                pltpu.make_async_copy(
                    bufs[s2], out_hbm.at[chunk_idx(k - 2)], out_sems[s2]
                ).wait()

                @pl.when(g < (BPW // NBUF) - 1)
                def _():
                    pltpu.async_copy(
                        embed_hbm.at[chunk_idx(k + 2)], bufs[s2], in_sems[s2]
                    )
        return carry

    lax.fori_loop(0, BPW // NBUF, group, 0)

    pltpu.make_async_copy(
        bufs[2], out_hbm.at[chunk_idx(BPW - 2)], out_sems[2]
    ).wait()
    pltpu.make_async_copy(
        bufs[3], out_hbm.at[chunk_idx(BPW - 1)], out_sems[3]
    ).wait()


def _tc_body(e_ref, p_ref, o_ref):
    o_ref[...] = e_ref[...] + p_ref[...][None, :, :]


def _tc_add(embed, pos_table):
    n = embed.shape[0]
    bb = 32
    return pl.pallas_call(
        _tc_body,
        grid=(n // bb,),
        in_specs=[
            pl.BlockSpec((bb, L, D), lambda i: (i, 0, 0)),
            pl.BlockSpec((L, D), lambda i: (0, 0)),
        ],
        out_specs=pl.BlockSpec((bb, L, D), lambda i: (i, 0, 0)),
        out_shape=jax.ShapeDtypeStruct((n, L, D), embed.dtype),
    )(embed, pos_table)


def kernel(embed, pos_table):
    e_sc = embed[B_TC:].reshape(B_SC * RG, CHUNK_ROWS, D)
    p_sc = pos_table.reshape(RG, CHUNK_ROWS, D)
    out_sc = _sc_add(e_sc, p_sc).reshape(B_SC, L, D)
    out_tc = _tc_add(embed[:B_TC], pos_table)
    return jnp.concatenate([out_tc, out_sc], axis=0)


# speedup vs baseline: 2.3646x; 2.3646x over previous
import functools
import jax
import jax.numpy as jnp
from jax import lax
from jax.experimental import pallas as pl
from jax.experimental.pallas import tpu as pltpu
from jax.experimental.pallas import tpu_sc as plsc

B, L, D = 1024, 512, 128
B_SC = 1024
B_TC = B - B_SC
RG = 4
CHUNK_ROWS = L // RG
NC, NS = 2, 16
NW = NC * NS
BGROUPS = NW // RG
BPW = B_SC // BGROUPS
NBUF = 4

_mesh = plsc.VectorSubcoreMesh(core_axis_name="c", subcore_axis_name="s")


@functools.partial(
    pl.kernel,
    mesh=_mesh,
    out_type=jax.ShapeDtypeStruct((B_SC * RG, CHUNK_ROWS, D), jnp.float32),
    scratch_types=(
        [pltpu.VMEM((CHUNK_ROWS, D), jnp.float32)]
        + [pltpu.VMEM((CHUNK_ROWS, D), jnp.float32) for _ in range(NBUF)]
        + [pltpu.SemaphoreType.DMA for _ in range(2 * NBUF)]
    ),
)
def _sc_add(embed_hbm, pos_hbm, out_hbm, pos_v, b0, b1, b2, b3,
            si0, si1, si2, si3, so0, so1, so2, so3):
    bufs = (b0, b1, b2, b3)
    in_sems = (si0, si1, si2, si3)
    out_sems = (so0, so1, so2, so3)

    wid = lax.axis_index("s") * NC + lax.axis_index("c")
    bg = wid // RG
    rg = wid % RG
    base = bg * BPW

    def chunk_idx(k):
        return (base + k) * RG + rg

    pltpu.sync_copy(pos_hbm.at[rg], pos_v)

    pltpu.async_copy(embed_hbm.at[chunk_idx(0)], bufs[0], in_sems[0])
    pltpu.async_copy(embed_hbm.at[chunk_idx(1)], bufs[1], in_sems[1])

    def group(g, carry):
        for s in range(NBUF):
            k = g * NBUF + s
            buf = bufs[s]
            c = chunk_idx(k)
            pltpu.make_async_copy(embed_hbm.at[c], buf, in_sems[s]).wait()

            def add_body(r, carry2):
                for j in range(D // 16):
                    sl = pl.ds(j * 16, 16)
                    plsc.addupdate(buf.at[r, sl], pos_v[r, sl])
                return carry2

            lax.fori_loop(0, CHUNK_ROWS, add_body, 0)

            pltpu.async_copy(buf, out_hbm.at[c], out_sems[s])

            s2 = (s + 2) % NBUF
            if s < 2:
                @pl.when(g >= 1)
                def _():
                    pltpu.make_async_copy(
                        bufs[s2], out_hbm.at[chunk_idx(k - 2)], out_sems[s2]
                    ).wait()

                pltpu.async_copy(
                    embed_hbm.at[chunk_idx(k + 2)], bufs[s2], in_sems[s2]
                )
            else:
                pltpu.make_async_copy(
                    bufs[s2], out_hbm.at[chunk_idx(k - 2)], out_sems[s2]
                ).wait()

                @pl.when(g < (BPW // NBUF) - 1)
                def _():
                    pltpu.async_copy(
                        embed_hbm.at[chunk_idx(k + 2)], bufs[s2], in_sems[s2]
                    )
        return carry

    lax.fori_loop(0, BPW // NBUF, group, 0)

    pltpu.make_async_copy(
        bufs[2], out_hbm.at[chunk_idx(BPW - 2)], out_sems[2]
    ).wait()
    pltpu.make_async_copy(
        bufs[3], out_hbm.at[chunk_idx(BPW - 1)], out_sems[3]
    ).wait()


def _tc_body(e_ref, p_ref, o_ref):
    o_ref[...] = e_ref[...] + p_ref[...][None, :, :]


def _tc_add(embed, pos_table):
    n = embed.shape[0]
    bb = 32
    return pl.pallas_call(
        _tc_body,
        grid=(n // bb,),
        in_specs=[
            pl.BlockSpec((bb, L, D), lambda i: (i, 0, 0)),
            pl.BlockSpec((L, D), lambda i: (0, 0)),
        ],
        out_specs=pl.BlockSpec((bb, L, D), lambda i: (i, 0, 0)),
        out_shape=jax.ShapeDtypeStruct((n, L, D), embed.dtype),
    )(embed, pos_table)


def kernel(embed, pos_table):
    e_sc = embed[B_TC:].reshape(B_SC * RG, CHUNK_ROWS, D)
    p_sc = pos_table.reshape(RG, CHUNK_ROWS, D)
    out_sc = _sc_add(e_sc, p_sc).reshape(B_SC, L, D)
    if B_TC == 0:
        return out_sc
    out_tc = _tc_add(embed[:B_TC], pos_table)
    return jnp.concatenate([out_tc, out_sc], axis=0)


# speedup vs baseline: 2.5038x; 1.0589x over previous
import functools
import jax
import jax.numpy as jnp
from jax import lax
from jax.experimental import pallas as pl
from jax.experimental.pallas import tpu as pltpu
from jax.experimental.pallas import tpu_sc as plsc

B, L, D = 1024, 512, 128
B_SC = 1024
B_TC = B - B_SC
RG = 4
CHUNK_ROWS = L // RG
NC, NS = 2, 16
NW = NC * NS
BGROUPS = NW // RG
BPW = B_SC // BGROUPS
NBUF = 4
PD = 3

_mesh = plsc.VectorSubcoreMesh(core_axis_name="c", subcore_axis_name="s")


@functools.partial(
    pl.kernel,
    mesh=_mesh,
    out_type=jax.ShapeDtypeStruct((B_SC * RG, CHUNK_ROWS, D), jnp.float32),
    scratch_types=(
        [pltpu.VMEM((CHUNK_ROWS, D), jnp.float32)]
        + [pltpu.VMEM((CHUNK_ROWS, D), jnp.float32) for _ in range(NBUF)]
        + [pltpu.SemaphoreType.DMA for _ in range(2 * NBUF)]
    ),
)
def _sc_add(embed_hbm, pos_hbm, out_hbm, pos_v, *rest):
    bufs = rest[:NBUF]
    in_sems = rest[NBUF:2 * NBUF]
    out_sems = rest[2 * NBUF:]

    wid = lax.axis_index("s") * NC + lax.axis_index("c")
    bg = wid // RG
    rg = wid % RG
    base = bg * BPW

    def chunk_idx(k):
        return (base + k) * RG + rg

    pltpu.sync_copy(pos_hbm.at[rg], pos_v)

    for j in range(PD):
        pltpu.async_copy(embed_hbm.at[chunk_idx(j)], bufs[j], in_sems[j])

    def group(g, carry):
        for s in range(NBUF):
            k = g * NBUF + s
            buf = bufs[s]
            c = chunk_idx(k)
            pltpu.make_async_copy(embed_hbm.at[c], buf, in_sems[s]).wait()

            def add_body(r, carry2):
                for j in range(D // 16):
                    sl = pl.ds(j * 16, 16)
                    plsc.addupdate(buf.at[r, sl], pos_v[r, sl])
                return carry2

            lax.fori_loop(0, CHUNK_ROWS, add_body, 0)

            pltpu.async_copy(buf, out_hbm.at[c], out_sems[s])

            sp = (s + PD) % NBUF
            kw = k + PD - NBUF
            if s < NBUF - PD:
                @pl.when(g >= 1)
                def _():
                    pltpu.make_async_copy(
                        bufs[sp], out_hbm.at[chunk_idx(kw)], out_sems[sp]
                    ).wait()

                pltpu.async_copy(
                    embed_hbm.at[chunk_idx(k + PD)], bufs[sp], in_sems[sp]
                )
            else:
                pltpu.make_async_copy(
                    bufs[sp], out_hbm.at[chunk_idx(kw)], out_sems[sp]
                ).wait()

                @pl.when(g < (BPW // NBUF) - 1)
                def _():
                    pltpu.async_copy(
                        embed_hbm.at[chunk_idx(k + PD)], bufs[sp], in_sems[sp]
                    )
        return carry

    lax.fori_loop(0, BPW // NBUF, group, 0)

    for j in range(NBUF - PD):
        kk = BPW - (NBUF - PD) + j
        pltpu.make_async_copy(
            bufs[kk % NBUF], out_hbm.at[chunk_idx(kk)], out_sems[kk % NBUF]
        ).wait()


def _tc_body(e_ref, p_ref, o_ref):
    o_ref[...] = e_ref[...] + p_ref[...][None, :, :]


def _tc_add(embed, pos_table):
    n = embed.shape[0]
    bb = 32
    return pl.pallas_call(
        _tc_body,
        grid=(n // bb,),
        in_specs=[
            pl.BlockSpec((bb, L, D), lambda i: (i, 0, 0)),
            pl.BlockSpec((L, D), lambda i: (0, 0)),
        ],
        out_specs=pl.BlockSpec((bb, L, D), lambda i: (i, 0, 0)),
        out_shape=jax.ShapeDtypeStruct((n, L, D), embed.dtype),
    )(embed, pos_table)


def kernel(embed, pos_table):
    e_sc = embed[B_TC:].reshape(B_SC * RG, CHUNK_ROWS, D)
    p_sc = pos_table.reshape(RG, CHUNK_ROWS, D)
    out_sc = _sc_add(e_sc, p_sc).reshape(B_SC, L, D)
    if B_TC == 0:
        return out_sc
    out_tc = _tc_add(embed[:B_TC], pos_table)
    return jnp.concatenate([out_tc, out_sc], axis=0)
